# Initial kernel scaffold; baseline (speedup 1.0000x reference)
import jax
import jax.numpy as jnp
from jax.experimental import pallas as pl


def kernel(scalar):
    raise NotImplementedError("write your pallas kernel here")


# speedup vs baseline: 7.3391x; 7.3391x over previous
import functools

import jax
import jax.numpy as jnp
from jax import lax
from jax.experimental import pallas as pl
from jax.experimental.pallas import tpu as pltpu
from jax.experimental.pallas import tpu_sc as plsc

V_MIN = -10.0
V_MAX = 10.0
ATOMS = 51
DZ = (V_MAX - V_MIN) / (ATOMS - 1)
N = 1048576

NC = 2
NS = 16
NW = NC * NS
ROWS_W = N // NW
C = 512
NCHUNK = ROWS_W // C
G = C // 16

_mesh = plsc.VectorSubcoreMesh(
    core_axis_name="c", subcore_axis_name="s", num_cores=NC, num_subcores=NS
)


@functools.partial(
    pl.kernel,
    out_type=jax.ShapeDtypeStruct((N * ATOMS,), jnp.float32),
    mesh=_mesh,
    scratch_types=[
        pltpu.VMEM((C,), jnp.float32),
        pltpu.VMEM((C * ATOMS,), jnp.float32),
    ],
    compiler_params=pltpu.CompilerParams(needs_layout_passes=False),
)
def _c51_sc(s_hbm, out_hbm, s_v, o_v):
    wid = lax.axis_index("s") * NC + lax.axis_index("c")
    base = wid * ROWS_W
    lanes51 = lax.iota(jnp.int32, 16) * ATOMS

    @pl.loop(0, NCHUNK)
    def _chunk(c):
        row0 = base + c * C
        pltpu.sync_copy(s_hbm.at[pl.ds(row0, C)], s_v)

        @pl.loop(0, G)
        def _group(g):
            sv = s_v[pl.ds(g * 16, 16)]
            t = jnp.minimum(jnp.maximum(sv, V_MIN), V_MAX)
            b = (t - V_MIN) / jnp.float32(DZ)
            idx0 = lanes51 + g * (16 * ATOMS)
            for j in range(ATOMS):
                v = jnp.maximum(1.0 - jnp.abs(b - jnp.float32(j)), 0.0)
                plsc.store_scatter(o_v, [idx0 + j], v)

        pltpu.sync_copy(o_v, out_hbm.at[pl.ds(row0 * ATOMS, C * ATOMS)])


def kernel(scalar):
    return _c51_sc(scalar).reshape(N, ATOMS)
